# baseline (speedup 1.0000x reference)
import functools

import jax
import jax.numpy as jnp
from jax import lax
from jax.experimental import pallas as pl
from jax.experimental.pallas import tpu as pltpu
from jax.experimental.pallas import tpu_sc as plsc

_N = 10000
_K = 32
_D = 128

_BP = 327680
_CH = 128
_NCH0 = 40
_NCH1 = 120
_BPW0 = _NCH0 * _CH
_BPW1 = _NCH1 * _CH
_BP0 = 16 * _BPW0
_SLOW_C = 0

_BN = 80
_NBLK = _N // _BN
_R = _BN * _K


def _silu(x):
    return x * (1.0 / (1.0 + jnp.exp(-x)))


def _ln_rows(x, g, b):
    m = jnp.mean(x, axis=-1, keepdims=True)
    xc = x - m
    v = jnp.mean(xc * xc, axis=-1, keepdims=True)
    return xc * lax.rsqrt(v + 1e-5) * g + b


def _dot(a, b):
    return jnp.dot(a, b, preferred_element_type=jnp.float32)


_NBUF = 5


def _sc_gather(table, idx_pad):
    mesh = plsc.VectorSubcoreMesh(core_axis_name="c", subcore_axis_name="s")

    @functools.partial(
        pl.kernel,
        mesh=mesh,
        out_type=jax.ShapeDtypeStruct((_BP, _D), jnp.float32),
        scratch_types=(
            [pltpu.VMEM((_BPW1,), jnp.int32),
             pltpu.VMEM((_NBUF, _CH, _D), jnp.float32)]
            + [pltpu.SemaphoreType.DMA] * (2 * _NBUF)
        ),
    )
    def gather_kernel(table_hbm, idx_hbm, out_hbm, idx_v, bufs, *sems):
        gsems, wsems = sems[:_NBUF], sems[_NBUF:]
        c = lax.axis_index("c")
        s = lax.axis_index("s")
        slow = c == _SLOW_C
        base = jnp.where(slow, s * _BPW0, _BP0 + s * _BPW1)
        base = pl.multiple_of(base, _CH)
        ngrp = jnp.where(slow, _NCH0 // _NBUF, _NCH1 // _NBUF)
        nch = jnp.where(slow, _NCH0, _NCH1)
        pltpu.sync_copy(idx_hbm.at[pl.ds(base, _BPW1)], idx_v)

        def group(g, carry):
            @pl.when(g < ngrp)
            def _run():
                c0 = g * _NBUF
                for b in range(_NBUF):
                    o = pl.multiple_of((c0 + b) * _CH, _CH)

                    @pl.when(g > 0)
                    def _wait_prev_write(o=o, b=b):
                        pltpu.make_async_copy(
                            bufs.at[b], out_hbm.at[pl.ds(base + o, _CH)],
                            wsems[b]).wait()

                    pltpu.async_copy(
                        table_hbm.at[idx_v.at[pl.ds(o, _CH)]], bufs.at[b],
                        gsems[b])
                for b in range(_NBUF):
                    o = pl.multiple_of((c0 + b) * _CH, _CH)
                    pltpu.make_async_copy(
                        table_hbm.at[idx_v.at[pl.ds(o, _CH)]], bufs.at[b],
                        gsems[b]).wait()
                    pltpu.async_copy(
                        bufs.at[b], out_hbm.at[pl.ds(base + o, _CH)],
                        wsems[b])
            return carry

        lax.fori_loop(0, _NCH1 // _NBUF, group, 0)
        for b in range(_NBUF):
            o = pl.multiple_of((nch - _NBUF + b) * _CH, _CH)
            pltpu.make_async_copy(
                bufs.at[b], out_hbm.at[pl.ds(base + o, _CH)], wsems[b]).wait()

    return gather_kernel(table, idx_pad)


def _precompute_p(nodes, w1b):
    def body(nodes_ref, w_ref, out_ref):
        out_ref[...] = _dot(nodes_ref[...], w_ref[...])

    return pl.pallas_call(
        body,
        out_shape=jax.ShapeDtypeStruct((_N, _D), jnp.float32),
    )(nodes, w1b)


def _full(shp):
    return pl.BlockSpec(shp, lambda i: tuple(0 for _ in shp))


def _stage1(nodes, pg, edges2d, mask2d, w1a, b1, w1c, w2, b2, w3, b3, w4, b4,
            f1, fb1, f2, fb2, g1, gb1, g2, gb2, e1b):
    def body(nodes_ref, pg_ref, edges_ref, mask_ref, w1a_ref, b1_ref, w1c_ref,
             w2_ref, b2_ref, w3_ref, b3_ref, w4_ref, b4_ref,
             f1_ref, fb1_ref, f2_ref, fb2_ref, g1_ref, gb1_ref, g2_ref,
             gb2_ref, e1b_ref, nodes_out, q_out):
        nb = nodes_ref[...]
        a1 = _dot(nb, w1a_ref[...]) + b1_ref[...]
        x = pg_ref[...].astype(jnp.float32) + _dot(edges_ref[...],
                                                   w1c_ref[...])
        x = x.reshape(_BN, _K, _D) + a1[:, None, :]
        h = _silu(x).reshape(_R, _D)
        h = _silu(_dot(h, w2_ref[...]) + b2_ref[...])
        h = _silu(_dot(h, w3_ref[...]) + b3_ref[...])
        h = _dot(h, w4_ref[...]) + b4_ref[...]
        s = jnp.sum((h * mask_ref[...]).reshape(_BN, _K, _D), axis=1)
        x1 = _ln_rows(nb + s, g1_ref[...], gb1_ref[...])
        hid = _silu(_dot(x1, f1_ref[...]) + fb1_ref[...])
        y = _ln_rows(x1 + _dot(hid, f2_ref[...]) + fb2_ref[...],
                     g2_ref[...], gb2_ref[...])
        nodes_out[...] = y
        q_out[...] = _dot(y, e1b_ref[...])

    return pl.pallas_call(
        body,
        grid=(_NBLK,),
        in_specs=[
            pl.BlockSpec((_BN, _D), lambda i: (i, 0)),
            pl.BlockSpec((_R, _D), lambda i: (i, 0)),
            pl.BlockSpec((_R, _D), lambda i: (i, 0)),
            pl.BlockSpec((_R, 1), lambda i: (i, 0)),
            _full((_D, _D)), _full((1, _D)), _full((_D, _D)),
            _full((_D, _D)), _full((1, _D)),
            _full((_D, _D)), _full((1, _D)),
            _full((_D, _D)), _full((1, _D)),
            _full((_D, 4 * _D)), _full((1, 4 * _D)),
            _full((4 * _D, _D)), _full((1, _D)),
            _full((1, _D)), _full((1, _D)), _full((1, _D)), _full((1, _D)),
            _full((_D, _D)),
        ],
        out_specs=[
            pl.BlockSpec((_BN, _D), lambda i: (i, 0)),
            pl.BlockSpec((_BN, _D), lambda i: (i, 0)),
        ],
        out_shape=[
            jax.ShapeDtypeStruct((_N, _D), jnp.float32),
            jax.ShapeDtypeStruct((_N, _D), jnp.float32),
        ],
    )(nodes, pg, edges2d, mask2d, w1a, b1, w1c, w2, b2, w3, b3, w4, b4,
      f1, fb1, f2, fb2, g1, gb1, g2, gb2, e1b)


def _stage2(nodes1, qg, edges2d, e1a, eb1, e1c, w2, b2, w3, b3, w4, b4,
            ge, gbe):
    def body(nodes_ref, qg_ref, edges_ref, e1a_ref, eb1_ref, e1c_ref,
             w2_ref, b2_ref, w3_ref, b3_ref, w4_ref, b4_ref,
             ge_ref, gbe_ref, edges_out):
        yb = nodes_ref[...]
        a2 = _dot(yb, e1a_ref[...]) + eb1_ref[...]
        eb = edges_ref[...]
        x = qg_ref[...].astype(jnp.float32) + _dot(eb, e1c_ref[...])
        x = x.reshape(_BN, _K, _D) + a2[:, None, :]
        h = _silu(x).reshape(_R, _D)
        h = _silu(_dot(h, w2_ref[...]) + b2_ref[...])
        h = _silu(_dot(h, w3_ref[...]) + b3_ref[...])
        h = _dot(h, w4_ref[...]) + b4_ref[...]
        edges_out[...] = _ln_rows(eb + h, ge_ref[...], gbe_ref[...])

    return pl.pallas_call(
        body,
        grid=(_NBLK,),
        in_specs=[
            pl.BlockSpec((_BN, _D), lambda i: (i, 0)),
            pl.BlockSpec((_R, _D), lambda i: (i, 0)),
            pl.BlockSpec((_R, _D), lambda i: (i, 0)),
            _full((_D, _D)), _full((1, _D)), _full((_D, _D)),
            _full((_D, _D)), _full((1, _D)),
            _full((_D, _D)), _full((1, _D)),
            _full((_D, _D)), _full((1, _D)),
            _full((1, _D)), _full((1, _D)),
        ],
        out_specs=[pl.BlockSpec((_R, _D), lambda i: (i, 0))],
        out_shape=[jax.ShapeDtypeStruct((_N * _K, _D), jnp.float32)],
    )(nodes1, qg, edges2d, e1a, eb1, e1c, w2, b2, w3, b3, w4, b4, ge, gbe)[0]


def kernel(nodes, edges, nbrs, nbr_mask, node_mlp, ffn, edge_mlp, ln1, ln2,
           edge_ln):
    wn, bn = node_mlp
    wf, bf = ffn
    we, be = edge_mlp

    w1a, w1b, w1c = wn[0][:_D], wn[0][_D:2 * _D], wn[0][2 * _D:]
    e1a, e1b, e1c = we[0][:_D], we[0][_D:2 * _D], we[0][2 * _D:]

    def row(v):
        return v.reshape(1, -1)

    idx = nbrs.reshape(-1).astype(jnp.int32)
    idx_pad = jnp.pad(idx, (0, _BP - _N * _K))
    edges2d = edges.reshape(_N * _K, _D)
    mask2d = nbr_mask.reshape(_N * _K, 1)

    p = _precompute_p(nodes, w1b)
    pg = _sc_gather(p, idx_pad)
    nodes1, q = _stage1(
        nodes, pg, edges2d, mask2d,
        w1a, row(bn[0]), w1c, wn[1], row(bn[1]), wn[2], row(bn[2]),
        wn[3], row(bn[3]),
        wf[0], row(bf[0]), wf[1], row(bf[1]),
        row(ln1[0]), row(ln1[1]), row(ln2[0]), row(ln2[1]), e1b)
    qg = _sc_gather(q, idx_pad)
    edges_out = _stage2(
        nodes1, qg, edges2d,
        e1a, row(be[0]), e1c, we[1], row(be[1]), we[2], row(be[2]),
        we[3], row(be[3]), row(edge_ln[0]), row(edge_ln[1]))
    return nodes1, edges_out.reshape(_N, _K, _D)


# speedup vs baseline: 1.8629x; 1.8629x over previous
import functools

import jax
import jax.numpy as jnp
from jax import lax
from jax.experimental import pallas as pl
from jax.experimental.pallas import tpu as pltpu
from jax.experimental.pallas import tpu_sc as plsc

_N = 10000
_K = 32
_D = 128

_BP = 327680
_CH = 128
_NCH0 = 40
_NCH1 = 120
_BPW0 = _NCH0 * _CH
_BPW1 = _NCH1 * _CH
_BP0 = 16 * _BPW0
_SLOW_C = 1

_BN = 80
_NBLK = _N // _BN
_R = _BN * _K


def _silu(x):
    return x * (1.0 / (1.0 + jnp.exp(-x)))


def _ln_rows(x, g, b):
    m = jnp.mean(x, axis=-1, keepdims=True)
    xc = x - m
    v = jnp.mean(xc * xc, axis=-1, keepdims=True)
    return xc * lax.rsqrt(v + 1e-5) * g + b


def _dot(a, b):
    return jnp.dot(a, b, preferred_element_type=jnp.float32)


_NBUF = 5


def _sc_gather(table, idx_pad):
    mesh = plsc.VectorSubcoreMesh(core_axis_name="c", subcore_axis_name="s")

    @functools.partial(
        pl.kernel,
        mesh=mesh,
        out_type=jax.ShapeDtypeStruct((_BP, _D), jnp.float32),
        scratch_types=(
            [pltpu.VMEM((_BPW1,), jnp.int32),
             pltpu.VMEM((_NBUF, _CH, _D), jnp.float32)]
            + [pltpu.SemaphoreType.DMA] * (2 * _NBUF)
        ),
    )
    def gather_kernel(table_hbm, idx_hbm, out_hbm, idx_v, bufs, *sems):
        gsems, wsems = sems[:_NBUF], sems[_NBUF:]
        c = lax.axis_index("c")
        s = lax.axis_index("s")
        slow = c == _SLOW_C
        base = jnp.where(slow, s * _BPW0, _BP0 + s * _BPW1)
        base = pl.multiple_of(base, _CH)
        ngrp = jnp.where(slow, _NCH0 // _NBUF, _NCH1 // _NBUF)
        nch = jnp.where(slow, _NCH0, _NCH1)
        pltpu.sync_copy(idx_hbm.at[pl.ds(base, _BPW1)], idx_v)

        def group(g, carry):
            @pl.when(g < ngrp)
            def _run():
                c0 = g * _NBUF
                for b in range(_NBUF):
                    o = pl.multiple_of((c0 + b) * _CH, _CH)

                    @pl.when(g > 0)
                    def _wait_prev_write(o=o, b=b):
                        pltpu.make_async_copy(
                            bufs.at[b], out_hbm.at[pl.ds(base + o, _CH)],
                            wsems[b]).wait()

                    pltpu.async_copy(
                        table_hbm.at[idx_v.at[pl.ds(o, _CH)]], bufs.at[b],
                        gsems[b])
                for b in range(_NBUF):
                    o = pl.multiple_of((c0 + b) * _CH, _CH)
                    pltpu.make_async_copy(
                        table_hbm.at[idx_v.at[pl.ds(o, _CH)]], bufs.at[b],
                        gsems[b]).wait()
                    pltpu.async_copy(
                        bufs.at[b], out_hbm.at[pl.ds(base + o, _CH)],
                        wsems[b])
            return carry

        lax.fori_loop(0, _NCH1 // _NBUF, group, 0)
        for b in range(_NBUF):
            o = pl.multiple_of((nch - _NBUF + b) * _CH, _CH)
            pltpu.make_async_copy(
                bufs.at[b], out_hbm.at[pl.ds(base + o, _CH)], wsems[b]).wait()

    return gather_kernel(table, idx_pad)


def _precompute_p(nodes, w1b):
    def body(nodes_ref, w_ref, out_ref):
        out_ref[...] = _dot(nodes_ref[...], w_ref[...])

    return pl.pallas_call(
        body,
        out_shape=jax.ShapeDtypeStruct((_N, _D), jnp.float32),
    )(nodes, w1b)


def _full(shp):
    return pl.BlockSpec(shp, lambda i: tuple(0 for _ in shp))


def _stage1(nodes, pg, edges2d, mask2d, w1a, b1, w1c, w2, b2, w3, b3, w4, b4,
            f1, fb1, f2, fb2, g1, gb1, g2, gb2, e1b):
    def body(nodes_ref, pg_ref, edges_ref, mask_ref, w1a_ref, b1_ref, w1c_ref,
             w2_ref, b2_ref, w3_ref, b3_ref, w4_ref, b4_ref,
             f1_ref, fb1_ref, f2_ref, fb2_ref, g1_ref, gb1_ref, g2_ref,
             gb2_ref, e1b_ref, nodes_out, q_out):
        nb = nodes_ref[...]
        a1 = _dot(nb, w1a_ref[...]) + b1_ref[...]
        x = pg_ref[...].astype(jnp.float32) + _dot(edges_ref[...],
                                                   w1c_ref[...])
        x = x.reshape(_BN, _K, _D) + a1[:, None, :]
        h = _silu(x).reshape(_R, _D)
        h = _silu(_dot(h, w2_ref[...]) + b2_ref[...])
        h = _silu(_dot(h, w3_ref[...]) + b3_ref[...])
        h = _dot(h, w4_ref[...]) + b4_ref[...]
        s = jnp.sum((h * mask_ref[...]).reshape(_BN, _K, _D), axis=1)
        x1 = _ln_rows(nb + s, g1_ref[...], gb1_ref[...])
        hid = _silu(_dot(x1, f1_ref[...]) + fb1_ref[...])
        y = _ln_rows(x1 + _dot(hid, f2_ref[...]) + fb2_ref[...],
                     g2_ref[...], gb2_ref[...])
        nodes_out[...] = y
        q_out[...] = _dot(y, e1b_ref[...])

    return pl.pallas_call(
        body,
        grid=(_NBLK,),
        in_specs=[
            pl.BlockSpec((_BN, _D), lambda i: (i, 0)),
            pl.BlockSpec((_R, _D), lambda i: (i, 0)),
            pl.BlockSpec((_R, _D), lambda i: (i, 0)),
            pl.BlockSpec((_R, 1), lambda i: (i, 0)),
            _full((_D, _D)), _full((1, _D)), _full((_D, _D)),
            _full((_D, _D)), _full((1, _D)),
            _full((_D, _D)), _full((1, _D)),
            _full((_D, _D)), _full((1, _D)),
            _full((_D, 4 * _D)), _full((1, 4 * _D)),
            _full((4 * _D, _D)), _full((1, _D)),
            _full((1, _D)), _full((1, _D)), _full((1, _D)), _full((1, _D)),
            _full((_D, _D)),
        ],
        out_specs=[
            pl.BlockSpec((_BN, _D), lambda i: (i, 0)),
            pl.BlockSpec((_BN, _D), lambda i: (i, 0)),
        ],
        out_shape=[
            jax.ShapeDtypeStruct((_N, _D), jnp.float32),
            jax.ShapeDtypeStruct((_N, _D), jnp.float32),
        ],
    )(nodes, pg, edges2d, mask2d, w1a, b1, w1c, w2, b2, w3, b3, w4, b4,
      f1, fb1, f2, fb2, g1, gb1, g2, gb2, e1b)


def _stage2(nodes1, qg, edges2d, e1a, eb1, e1c, w2, b2, w3, b3, w4, b4,
            ge, gbe):
    def body(nodes_ref, qg_ref, edges_ref, e1a_ref, eb1_ref, e1c_ref,
             w2_ref, b2_ref, w3_ref, b3_ref, w4_ref, b4_ref,
             ge_ref, gbe_ref, edges_out):
        yb = nodes_ref[...]
        a2 = _dot(yb, e1a_ref[...]) + eb1_ref[...]
        eb = edges_ref[...]
        x = qg_ref[...].astype(jnp.float32) + _dot(eb, e1c_ref[...])
        x = x.reshape(_BN, _K, _D) + a2[:, None, :]
        h = _silu(x).reshape(_R, _D)
        h = _silu(_dot(h, w2_ref[...]) + b2_ref[...])
        h = _silu(_dot(h, w3_ref[...]) + b3_ref[...])
        h = _dot(h, w4_ref[...]) + b4_ref[...]
        edges_out[...] = _ln_rows(eb + h, ge_ref[...], gbe_ref[...])

    return pl.pallas_call(
        body,
        grid=(_NBLK,),
        in_specs=[
            pl.BlockSpec((_BN, _D), lambda i: (i, 0)),
            pl.BlockSpec((_R, _D), lambda i: (i, 0)),
            pl.BlockSpec((_R, _D), lambda i: (i, 0)),
            _full((_D, _D)), _full((1, _D)), _full((_D, _D)),
            _full((_D, _D)), _full((1, _D)),
            _full((_D, _D)), _full((1, _D)),
            _full((_D, _D)), _full((1, _D)),
            _full((1, _D)), _full((1, _D)),
        ],
        out_specs=[pl.BlockSpec((_R, _D), lambda i: (i, 0))],
        out_shape=[jax.ShapeDtypeStruct((_N * _K, _D), jnp.float32)],
    )(nodes1, qg, edges2d, e1a, eb1, e1c, w2, b2, w3, b3, w4, b4, ge, gbe)[0]


def kernel(nodes, edges, nbrs, nbr_mask, node_mlp, ffn, edge_mlp, ln1, ln2,
           edge_ln):
    wn, bn = node_mlp
    wf, bf = ffn
    we, be = edge_mlp

    w1a, w1b, w1c = wn[0][:_D], wn[0][_D:2 * _D], wn[0][2 * _D:]
    e1a, e1b, e1c = we[0][:_D], we[0][_D:2 * _D], we[0][2 * _D:]

    def row(v):
        return v.reshape(1, -1)

    idx = nbrs.reshape(-1).astype(jnp.int32)
    pad_idx = jnp.arange(_BP - _N * _K, dtype=jnp.int32) % _N
    idx_pad = jnp.concatenate([idx, pad_idx])
    edges2d = edges.reshape(_N * _K, _D)
    mask2d = nbr_mask.reshape(_N * _K, 1)

    p = _precompute_p(nodes, w1b)
    pg = _sc_gather(p, idx_pad)
    nodes1, q = _stage1(
        nodes, pg, edges2d, mask2d,
        w1a, row(bn[0]), w1c, wn[1], row(bn[1]), wn[2], row(bn[2]),
        wn[3], row(bn[3]),
        wf[0], row(bf[0]), wf[1], row(bf[1]),
        row(ln1[0]), row(ln1[1]), row(ln2[0]), row(ln2[1]), e1b)
    qg = _sc_gather(q, idx_pad)
    edges_out = _stage2(
        nodes1, qg, edges2d,
        e1a, row(be[0]), e1c, we[1], row(be[1]), we[2], row(be[2]),
        we[3], row(be[3]), row(edge_ln[0]), row(edge_ln[1]))
    return nodes1, edges_out.reshape(_N, _K, _D)


# speedup vs baseline: 1.9128x; 1.0268x over previous
import functools

import jax
import jax.numpy as jnp
from jax import lax
from jax.experimental import pallas as pl
from jax.experimental.pallas import tpu as pltpu
from jax.experimental.pallas import tpu_sc as plsc

_N = 10000
_K = 32
_D = 128

_BP = 327680
_CH = 128
_NCH0 = 80
_NCH1 = 80
_BPW0 = _NCH0 * _CH
_BPW1 = _NCH1 * _CH
_BP0 = 16 * _BPW0
_SLOW_C = 1

_BN = 80
_NBLK = _N // _BN
_R = _BN * _K


def _silu(x):
    return x * (1.0 / (1.0 + jnp.exp(-x)))


def _ln_rows(x, g, b):
    m = jnp.mean(x, axis=-1, keepdims=True)
    xc = x - m
    v = jnp.mean(xc * xc, axis=-1, keepdims=True)
    return xc * lax.rsqrt(v + 1e-5) * g + b


def _dot(a, b):
    return jnp.dot(a, b, preferred_element_type=jnp.float32)


_NBUF = 5


def _sc_gather(table, idx_pad):
    mesh = plsc.VectorSubcoreMesh(core_axis_name="c", subcore_axis_name="s")

    @functools.partial(
        pl.kernel,
        mesh=mesh,
        out_type=jax.ShapeDtypeStruct((_BP, _D), jnp.float32),
        scratch_types=(
            [pltpu.VMEM((_BPW1,), jnp.int32),
             pltpu.VMEM((_NBUF, _CH, _D), jnp.float32)]
            + [pltpu.SemaphoreType.DMA] * (2 * _NBUF)
        ),
    )
    def gather_kernel(table_hbm, idx_hbm, out_hbm, idx_v, bufs, *sems):
        gsems, wsems = sems[:_NBUF], sems[_NBUF:]
        c = lax.axis_index("c")
        s = lax.axis_index("s")
        slow = c == _SLOW_C
        base = jnp.where(slow, s * _BPW0, _BP0 + s * _BPW1)
        base = pl.multiple_of(base, _CH)
        ngrp = jnp.where(slow, _NCH0 // _NBUF, _NCH1 // _NBUF)
        nch = jnp.where(slow, _NCH0, _NCH1)
        pltpu.sync_copy(idx_hbm.at[pl.ds(base, _BPW1)], idx_v)

        def group(g, carry):
            @pl.when(g < ngrp)
            def _run():
                c0 = g * _NBUF
                for b in range(_NBUF):
                    o = pl.multiple_of((c0 + b) * _CH, _CH)

                    @pl.when(g > 0)
                    def _wait_prev_write(o=o, b=b):
                        pltpu.make_async_copy(
                            bufs.at[b], out_hbm.at[pl.ds(base + o, _CH)],
                            wsems[b]).wait()

                    pltpu.async_copy(
                        table_hbm.at[idx_v.at[pl.ds(o, _CH)]], bufs.at[b],
                        gsems[b])
                for b in range(_NBUF):
                    o = pl.multiple_of((c0 + b) * _CH, _CH)
                    pltpu.make_async_copy(
                        table_hbm.at[idx_v.at[pl.ds(o, _CH)]], bufs.at[b],
                        gsems[b]).wait()
                    pltpu.async_copy(
                        bufs.at[b], out_hbm.at[pl.ds(base + o, _CH)],
                        wsems[b])
            return carry

        lax.fori_loop(0, _NCH1 // _NBUF, group, 0)
        for b in range(_NBUF):
            o = pl.multiple_of((nch - _NBUF + b) * _CH, _CH)
            pltpu.make_async_copy(
                bufs.at[b], out_hbm.at[pl.ds(base + o, _CH)], wsems[b]).wait()

    return gather_kernel(table, idx_pad)


def _precompute_p(nodes, w1b):
    def body(nodes_ref, w_ref, out_ref):
        out_ref[...] = _dot(nodes_ref[...], w_ref[...])

    return pl.pallas_call(
        body,
        out_shape=jax.ShapeDtypeStruct((_N, _D), jnp.float32),
    )(nodes, w1b)


def _full(shp):
    return pl.BlockSpec(shp, lambda i: tuple(0 for _ in shp))


def _stage1(nodes, pg, edges2d, mask2d, w1a, b1, w1c, w2, b2, w3, b3, w4, b4,
            f1, fb1, f2, fb2, g1, gb1, g2, gb2, e1b):
    def body(nodes_ref, pg_ref, edges_ref, mask_ref, w1a_ref, b1_ref, w1c_ref,
             w2_ref, b2_ref, w3_ref, b3_ref, w4_ref, b4_ref,
             f1_ref, fb1_ref, f2_ref, fb2_ref, g1_ref, gb1_ref, g2_ref,
             gb2_ref, e1b_ref, nodes_out, q_out):
        nb = nodes_ref[...]
        a1 = _dot(nb, w1a_ref[...]) + b1_ref[...]
        x = pg_ref[...].astype(jnp.float32) + _dot(edges_ref[...],
                                                   w1c_ref[...])
        x = x.reshape(_BN, _K, _D) + a1[:, None, :]
        h = _silu(x).reshape(_R, _D)
        h = _silu(_dot(h, w2_ref[...]) + b2_ref[...])
        h = _silu(_dot(h, w3_ref[...]) + b3_ref[...])
        h = _dot(h, w4_ref[...]) + b4_ref[...]
        s = jnp.sum((h * mask_ref[...]).reshape(_BN, _K, _D), axis=1)
        x1 = _ln_rows(nb + s, g1_ref[...], gb1_ref[...])
        hid = _silu(_dot(x1, f1_ref[...]) + fb1_ref[...])
        y = _ln_rows(x1 + _dot(hid, f2_ref[...]) + fb2_ref[...],
                     g2_ref[...], gb2_ref[...])
        nodes_out[...] = y
        q_out[...] = _dot(y, e1b_ref[...])

    return pl.pallas_call(
        body,
        grid=(_NBLK,),
        in_specs=[
            pl.BlockSpec((_BN, _D), lambda i: (i, 0)),
            pl.BlockSpec((_R, _D), lambda i: (i, 0)),
            pl.BlockSpec((_R, _D), lambda i: (i, 0)),
            pl.BlockSpec((_R, 1), lambda i: (i, 0)),
            _full((_D, _D)), _full((1, _D)), _full((_D, _D)),
            _full((_D, _D)), _full((1, _D)),
            _full((_D, _D)), _full((1, _D)),
            _full((_D, _D)), _full((1, _D)),
            _full((_D, 4 * _D)), _full((1, 4 * _D)),
            _full((4 * _D, _D)), _full((1, _D)),
            _full((1, _D)), _full((1, _D)), _full((1, _D)), _full((1, _D)),
            _full((_D, _D)),
        ],
        out_specs=[
            pl.BlockSpec((_BN, _D), lambda i: (i, 0)),
            pl.BlockSpec((_BN, _D), lambda i: (i, 0)),
        ],
        out_shape=[
            jax.ShapeDtypeStruct((_N, _D), jnp.float32),
            jax.ShapeDtypeStruct((_N, _D), jnp.float32),
        ],
    )(nodes, pg, edges2d, mask2d, w1a, b1, w1c, w2, b2, w3, b3, w4, b4,
      f1, fb1, f2, fb2, g1, gb1, g2, gb2, e1b)


def _stage2(nodes1, qg, edges2d, e1a, eb1, e1c, w2, b2, w3, b3, w4, b4,
            ge, gbe):
    def body(nodes_ref, qg_ref, edges_ref, e1a_ref, eb1_ref, e1c_ref,
             w2_ref, b2_ref, w3_ref, b3_ref, w4_ref, b4_ref,
             ge_ref, gbe_ref, edges_out):
        yb = nodes_ref[...]
        a2 = _dot(yb, e1a_ref[...]) + eb1_ref[...]
        eb = edges_ref[...]
        x = qg_ref[...].astype(jnp.float32) + _dot(eb, e1c_ref[...])
        x = x.reshape(_BN, _K, _D) + a2[:, None, :]
        h = _silu(x).reshape(_R, _D)
        h = _silu(_dot(h, w2_ref[...]) + b2_ref[...])
        h = _silu(_dot(h, w3_ref[...]) + b3_ref[...])
        h = _dot(h, w4_ref[...]) + b4_ref[...]
        edges_out[...] = _ln_rows(eb + h, ge_ref[...], gbe_ref[...])

    return pl.pallas_call(
        body,
        grid=(_NBLK,),
        in_specs=[
            pl.BlockSpec((_BN, _D), lambda i: (i, 0)),
            pl.BlockSpec((_R, _D), lambda i: (i, 0)),
            pl.BlockSpec((_R, _D), lambda i: (i, 0)),
            _full((_D, _D)), _full((1, _D)), _full((_D, _D)),
            _full((_D, _D)), _full((1, _D)),
            _full((_D, _D)), _full((1, _D)),
            _full((_D, _D)), _full((1, _D)),
            _full((1, _D)), _full((1, _D)),
        ],
        out_specs=[pl.BlockSpec((_R, _D), lambda i: (i, 0))],
        out_shape=[jax.ShapeDtypeStruct((_N * _K, _D), jnp.float32)],
    )(nodes1, qg, edges2d, e1a, eb1, e1c, w2, b2, w3, b3, w4, b4, ge, gbe)[0]


def kernel(nodes, edges, nbrs, nbr_mask, node_mlp, ffn, edge_mlp, ln1, ln2,
           edge_ln):
    wn, bn = node_mlp
    wf, bf = ffn
    we, be = edge_mlp

    w1a, w1b, w1c = wn[0][:_D], wn[0][_D:2 * _D], wn[0][2 * _D:]
    e1a, e1b, e1c = we[0][:_D], we[0][_D:2 * _D], we[0][2 * _D:]

    def row(v):
        return v.reshape(1, -1)

    idx = nbrs.reshape(-1).astype(jnp.int32)
    pad_idx = jnp.arange(_BP - _N * _K, dtype=jnp.int32) % _N
    idx_pad = jnp.concatenate([idx, pad_idx])
    edges2d = edges.reshape(_N * _K, _D)
    mask2d = nbr_mask.reshape(_N * _K, 1)

    p = _precompute_p(nodes, w1b)
    pg = _sc_gather(p, idx_pad)
    nodes1, q = _stage1(
        nodes, pg, edges2d, mask2d,
        w1a, row(bn[0]), w1c, wn[1], row(bn[1]), wn[2], row(bn[2]),
        wn[3], row(bn[3]),
        wf[0], row(bf[0]), wf[1], row(bf[1]),
        row(ln1[0]), row(ln1[1]), row(ln2[0]), row(ln2[1]), e1b)
    qg = _sc_gather(q, idx_pad)
    edges_out = _stage2(
        nodes1, qg, edges2d,
        e1a, row(be[0]), e1c, we[1], row(be[1]), we[2], row(be[2]),
        we[3], row(be[3]), row(edge_ln[0]), row(edge_ln[1]))
    return nodes1, edges_out.reshape(_N, _K, _D)
